# baseline (speedup 1.0000x reference)
import functools

import jax
import jax.numpy as jnp
from jax import lax
from jax.experimental import pallas as pl
from jax.experimental.pallas import tpu as pltpu
from jax.experimental.pallas import tpu_sc as plsc

_VOCAB = 1000000
_MAXLEN = 200
_EMBED = 64
_BATCH = 1024

_INFO = plsc.get_sparse_core_info()
_NC, _NS, _L = _INFO.num_cores, _INFO.num_subcores, _INFO.num_lanes
_NW = _NC * _NS
_ROWS_TOTAL = _BATCH * _MAXLEN
_ROWS_PER_W = _ROWS_TOTAL // _NW
_SEQ_PER_W = _ROWS_PER_W // _MAXLEN
_VECS_PER_ROW = _EMBED // _L


def _body(x_hbm, tok_hbm, pos_hbm, out_hbm, idx_v, pos_v, buf_v, sem):
    wid = lax.axis_index("s") * _NC + lax.axis_index("c")
    base = wid * _ROWS_PER_W

    pltpu.sync_copy(x_hbm.at[pl.ds(base, _ROWS_PER_W)], idx_v)
    pltpu.sync_copy(pos_hbm, pos_v)

    def chunk_body(c, carry):
        row0 = c * _MAXLEN
        pltpu.async_copy(
            tok_hbm.at[idx_v.at[pl.ds(row0, _MAXLEN)]], buf_v, sem
        ).wait()

        def add_row(r, carry2):
            for v in range(_VECS_PER_ROW):
                sl = pl.ds(v * _L, _L)
                buf_v[r, sl] = buf_v[r, sl] + pos_v[r, sl]
            return carry2

        lax.fori_loop(0, _MAXLEN, add_row, 0, unroll=2)

        pltpu.sync_copy(buf_v, out_hbm.at[pl.ds(base + row0, _MAXLEN)])
        return carry

    lax.fori_loop(0, _SEQ_PER_W, chunk_body, 0)


@jax.jit
def _run(x_flat, token_emb, pos_emb):
    mesh = plsc.VectorSubcoreMesh(core_axis_name="c", subcore_axis_name="s")
    k = functools.partial(
        pl.kernel,
        mesh=mesh,
        out_type=jax.ShapeDtypeStruct((_ROWS_TOTAL, _EMBED), jnp.float32),
        scratch_types=[
            pltpu.VMEM((_ROWS_PER_W,), jnp.int32),
            pltpu.VMEM((_MAXLEN, _EMBED), jnp.float32),
            pltpu.VMEM((_MAXLEN, _EMBED), jnp.float32),
            pltpu.SemaphoreType.DMA,
        ],
        compiler_params=pltpu.CompilerParams(use_tc_tiling_on_sc=False),
    )(_body)
    return k(x_flat, token_emb, pos_emb)


def kernel(x, token_emb, pos_emb):
    x_flat = x.reshape(-1).astype(jnp.int32)
    out = _run(x_flat, token_emb, pos_emb)
    return out.reshape(_BATCH, _MAXLEN, _EMBED)


# speedup vs baseline: 1.1783x; 1.1783x over previous
import functools

import jax
import jax.numpy as jnp
from jax import lax
from jax.experimental import pallas as pl
from jax.experimental.pallas import tpu as pltpu
from jax.experimental.pallas import tpu_sc as plsc

_VOCAB = 1000000
_MAXLEN = 200
_EMBED = 64
_BATCH = 1024

_INFO = plsc.get_sparse_core_info()
_NC, _NS, _L = _INFO.num_cores, _INFO.num_subcores, _INFO.num_lanes
_NW = _NC * _NS
_ROWS_TOTAL = _BATCH * _MAXLEN
_ROWS_PER_W = _ROWS_TOTAL // _NW
_SEQ_PER_W = _ROWS_PER_W // _MAXLEN
_VECS_PER_ROW = _EMBED // _L
_NBUF = 8
_PREF = 4
_ROUNDS = _SEQ_PER_W // _NBUF


def _body(x_hbm, tok_hbm, pos_hbm, out_hbm, idx_v, pos_v, bufs, gsem, osem):
    wid = lax.axis_index("s") * _NC + lax.axis_index("c")
    base = wid * _ROWS_PER_W

    pltpu.sync_copy(x_hbm.at[pl.ds(base, _ROWS_PER_W)], idx_v)
    pltpu.sync_copy(pos_hbm, pos_v)

    def g_copy(c, slot):
        return pltpu.make_async_copy(
            tok_hbm.at[idx_v.at[pl.ds(c * _MAXLEN, _MAXLEN)]],
            bufs[slot],
            gsem.at[slot],
        )

    def o_copy(c, slot):
        return pltpu.make_async_copy(
            bufs[slot],
            out_hbm.at[pl.ds(base + c * _MAXLEN, _MAXLEN)],
            osem.at[slot],
        )

    for b in range(_PREF):
        g_copy(b, b).start()

    def round_body(r, carry):
        for b in range(_NBUF):
            c = r * _NBUF + b
            g_copy(c, b).wait()

            def add_row(rr, carry2):
                for v in range(_VECS_PER_ROW):
                    sl = pl.ds(v * _L, _L)
                    plsc.addupdate(bufs[b].at[rr, sl], pos_v[rr, sl])
                return carry2

            lax.fori_loop(0, _MAXLEN, add_row, 0, unroll=4)

            o_copy(c, b).start()

            sp = (b + _PREF) % _NBUF
            if b < _PREF:
                @pl.when(r >= 1)
                def _():
                    o_copy(c - _PREF, sp).wait()
                    g_copy(c + _PREF, sp).start()

                @pl.when(r == 0)
                def _():
                    g_copy(c + _PREF, sp).start()
            else:
                o_copy(c - _PREF, sp).wait()

                @pl.when(r < _ROUNDS - 1)
                def _():
                    g_copy(c + _PREF, sp).start()

        return carry

    lax.fori_loop(0, _ROUNDS, round_body, 0)

    for b in range(_PREF, _NBUF):
        o_copy((_ROUNDS - 1) * _NBUF + b, b).wait()


@jax.jit
def _run(x_flat, token_emb, pos_emb):
    mesh = plsc.VectorSubcoreMesh(core_axis_name="c", subcore_axis_name="s")

    def wrapped(x_hbm, tok_hbm, pos_hbm, out_hbm, idx_v, pos_v, *rest):
        bufs = rest[:_NBUF]
        gsem, osem = rest[_NBUF:]
        _body(x_hbm, tok_hbm, pos_hbm, out_hbm, idx_v, pos_v, bufs, gsem, osem)

    k = functools.partial(
        pl.kernel,
        mesh=mesh,
        out_type=jax.ShapeDtypeStruct((_ROWS_TOTAL, _EMBED), jnp.float32),
        scratch_types=(
            [
                pltpu.VMEM((_ROWS_PER_W,), jnp.int32),
                pltpu.VMEM((_MAXLEN, _EMBED), jnp.float32),
            ]
            + [pltpu.VMEM((_MAXLEN, _EMBED), jnp.float32) for _ in range(_NBUF)]
            + [
                pltpu.SemaphoreType.DMA((_NBUF,)),
                pltpu.SemaphoreType.DMA((_NBUF,)),
            ]
        ),
        compiler_params=pltpu.CompilerParams(use_tc_tiling_on_sc=False),
    )(wrapped)
    return k(x_flat, token_emb, pos_emb)


def kernel(x, token_emb, pos_emb):
    x_flat = x.reshape(-1).astype(jnp.int32)
    out = _run(x_flat, token_emb, pos_emb)
    return out.reshape(_BATCH, _MAXLEN, _EMBED)
